# baseline (speedup 1.0000x reference)
import functools

import jax
import jax.numpy as jnp
from jax import lax
from jax.experimental import pallas as pl
from jax.experimental.pallas import tpu as pltpu
from jax.experimental.pallas import tpu_sc as plsc

_DIM = 768
_MAX_H = 32
_MAX_W = 32
_B = 64
_L = 1024
_N = _B * _L

_NC = 2
_NS = 16
_NW = _NC * _NS
_BPW = _N // _NW
_CH = 32
_NCH = _BPW // _CH
_NG = _NCH // 3
_LANES = 16
_GRP = _CH // _LANES


def _prep_body(hid_ref, wid_ref, h_ref, w_ref, cid_ref, tab_ref):
    cid_ref[...] = hid_ref[...] * _MAX_W + wid_ref[...]
    tab_ref[...] = h_ref[...][:, None, :] + w_ref[...][None, :, :]


def _prep(height_ids, width_ids, h_embed, w_embed):
    return pl.pallas_call(
        _prep_body,
        out_shape=[
            jax.ShapeDtypeStruct((_B, _L), jnp.int32),
            jax.ShapeDtypeStruct((_MAX_H, _MAX_W, _DIM), jnp.float32),
        ],
    )(height_ids, width_ids, h_embed, w_embed)


_sc_mesh = plsc.VectorSubcoreMesh(
    core_axis_name="c", subcore_axis_name="s", num_cores=_NC, num_subcores=_NS
)


@functools.partial(
    pl.kernel,
    mesh=_sc_mesh,
    compiler_params=pltpu.CompilerParams(needs_layout_passes=False),
    out_type=jax.ShapeDtypeStruct((_N, _DIM), jnp.float32),
    scratch_types=[
        pltpu.VMEM((_MAX_H * _DIM,), jnp.float32),
        pltpu.VMEM((_MAX_W * _DIM,), jnp.float32),
        pltpu.VMEM((_BPW,), jnp.int32),
        [pltpu.VMEM((_CH, _DIM), jnp.float32) for _ in range(3)],
        [pltpu.SemaphoreType.DMA for _ in range(3)],
        [pltpu.SemaphoreType.DMA for _ in range(3)],
    ],
)
def _sc_hybrid(tab_hbm, cid_hbm, htab_hbm, wtab_hbm, out_hbm, htab, wtab, cid_v, bufs, gs, ss):
    wid = lax.axis_index("s") * _NC + lax.axis_index("c")
    base = wid * _BPW
    pltpu.sync_copy(htab_hbm, htab)
    pltpu.sync_copy(wtab_hbm, wtab)
    pltpu.sync_copy(cid_hbm.at[pl.ds(base, _BPW)], cid_v)

    def g_src(c):
        return tab_hbm.at[cid_v.at[pl.ds(c * _CH, _CH)]]

    def o_dst(c):
        return out_hbm.at[pl.ds(base + c * _CH, _CH)]

    def expand_group(c, grp, buf):
        pos = c * _CH + grp * _LANES
        cv = cid_v[pl.ds(pos, _LANES)]
        hv = (cv >> 5) * _DIM
        wv = (cv & 31) * _DIM
        hbs = [hv[u] for u in range(_LANES)]
        wbs = [wv[u] for u in range(_LANES)]

        @plsc.parallel_loop(0, _DIM, step=_LANES, unroll=6)
        def _(d):
            for u in range(_LANES):
                a = htab[pl.ds(hbs[u] + d, _LANES)]
                b = wtab[pl.ds(wbs[u] + d, _LANES)]
                buf[grp * _LANES + u, pl.ds(d, _LANES)] = a + b

    def body(g, carry):
        c0 = 3 * g
        c1 = c0 + 1
        c2 = c0 + 2

        @pl.when(g > 0)
        def _():
            pltpu.make_async_copy(bufs[0], o_dst(c0 - 3), ss[0]).wait()

        pltpu.async_copy(g_src(c0), bufs[0], gs[0])

        @pl.when(g > 0)
        def _():
            pltpu.make_async_copy(bufs[1], o_dst(c1 - 3), ss[1]).wait()

        pltpu.async_copy(g_src(c1), bufs[1], gs[1])

        @pl.when(g > 0)
        def _():
            pltpu.make_async_copy(bufs[2], o_dst(c2 - 3), ss[2]).wait()

        expand_group(c2, 0, bufs[2])
        pltpu.make_async_copy(g_src(c0), bufs[0], gs[0]).wait()
        pltpu.async_copy(bufs[0], o_dst(c0), ss[0])
        expand_group(c2, 1, bufs[2])
        pltpu.make_async_copy(g_src(c1), bufs[1], gs[1]).wait()
        pltpu.async_copy(bufs[1], o_dst(c1), ss[1])
        pltpu.async_copy(bufs[2], o_dst(c2), ss[2])
        return carry

    lax.fori_loop(0, _NG, body, 0)

    last = _NCH - 1
    pltpu.make_async_copy(bufs[0], o_dst(last - 3), ss[0]).wait()
    pltpu.async_copy(g_src(last), bufs[0], gs[0])
    pltpu.make_async_copy(bufs[1], o_dst(last - 2), ss[1]).wait()
    pltpu.make_async_copy(bufs[2], o_dst(last - 1), ss[2]).wait()
    pltpu.make_async_copy(g_src(last), bufs[0], gs[0]).wait()
    pltpu.async_copy(bufs[0], o_dst(last), ss[0])
    pltpu.make_async_copy(bufs[0], o_dst(last), ss[0]).wait()


def kernel(height_ids, width_ids, h_embed, w_embed):
    hid = height_ids.astype(jnp.int32)
    wid = width_ids.astype(jnp.int32)
    cid, tab = _prep(hid, wid, h_embed, w_embed)
    out = _sc_hybrid(
        tab.reshape(_MAX_H * _MAX_W, _DIM),
        cid.reshape(_N),
        h_embed.reshape(_MAX_H * _DIM),
        w_embed.reshape(_MAX_W * _DIM),
    )
    return out.reshape(_B, _L, _DIM)


# speedup vs baseline: 1.0039x; 1.0039x over previous
import functools

import jax
import jax.numpy as jnp
from jax import lax
from jax.experimental import pallas as pl
from jax.experimental.pallas import tpu as pltpu
from jax.experimental.pallas import tpu_sc as plsc

_DIM = 768
_MAX_H = 32
_MAX_W = 32
_B = 64
_L = 1024
_N = _B * _L

_NC = 2
_NS = 16
_NW = _NC * _NS
_BPW = _N // _NW
_CH = 16
_NCH = _BPW // _CH
_NG = _NCH // 6
_LANES = 16
_GRP = _CH // _LANES


def _prep_body(hid_ref, wid_ref, h_ref, w_ref, cid_ref, tab_ref):
    cid_ref[...] = hid_ref[...] * _MAX_W + wid_ref[...]
    tab_ref[...] = h_ref[...][:, None, :] + w_ref[...][None, :, :]


def _prep(height_ids, width_ids, h_embed, w_embed):
    return pl.pallas_call(
        _prep_body,
        out_shape=[
            jax.ShapeDtypeStruct((_B, _L), jnp.int32),
            jax.ShapeDtypeStruct((_MAX_H, _MAX_W, _DIM), jnp.float32),
        ],
    )(height_ids, width_ids, h_embed, w_embed)


_sc_mesh = plsc.VectorSubcoreMesh(
    core_axis_name="c", subcore_axis_name="s", num_cores=_NC, num_subcores=_NS
)


@functools.partial(
    pl.kernel,
    mesh=_sc_mesh,
    compiler_params=pltpu.CompilerParams(needs_layout_passes=False),
    out_type=jax.ShapeDtypeStruct((_N, _DIM), jnp.float32),
    scratch_types=[
        pltpu.VMEM((_MAX_H * _DIM,), jnp.float32),
        pltpu.VMEM((_MAX_W * _DIM,), jnp.float32),
        pltpu.VMEM((_BPW,), jnp.int32),
        [pltpu.VMEM((_CH, _DIM), jnp.float32) for _ in range(6)],
        [pltpu.SemaphoreType.DMA for _ in range(6)],
        [pltpu.SemaphoreType.DMA for _ in range(6)],
    ],
)
def _sc_hybrid(tab_hbm, cid_hbm, htab_hbm, wtab_hbm, out_hbm, htab, wtab, cid_v, bufs, gs, ss):
    wid = lax.axis_index("s") * _NC + lax.axis_index("c")
    base = wid * _BPW
    pltpu.sync_copy(htab_hbm, htab)
    pltpu.sync_copy(wtab_hbm, wtab)
    pltpu.sync_copy(cid_hbm.at[pl.ds(base, _BPW)], cid_v)

    def g_src(c):
        return tab_hbm.at[cid_v.at[pl.ds(c * _CH, _CH)]]

    def o_dst(c):
        return out_hbm.at[pl.ds(base + c * _CH, _CH)]

    def expand_group(c, grp, buf):
        pos = c * _CH + grp * _LANES
        cv = cid_v[pl.ds(pos, _LANES)]
        hv = (cv >> 5) * _DIM
        wv = (cv & 31) * _DIM
        hbs = [hv[u] for u in range(_LANES)]
        wbs = [wv[u] for u in range(_LANES)]

        @plsc.parallel_loop(0, _DIM, step=_LANES, unroll=6)
        def _(d):
            for u in range(_LANES):
                a = htab[pl.ds(hbs[u] + d, _LANES)]
                b = wtab[pl.ds(wbs[u] + d, _LANES)]
                buf[grp * _LANES + u, pl.ds(d, _LANES)] = a + b

    def body(g, carry):
        for half in range(2):
            j = 3 * half
            c0 = 6 * g + j
            c1 = c0 + 1
            c2 = c0 + 2

            @pl.when(g > 0)
            def _(j=j, c0=c0):
                pltpu.make_async_copy(bufs[j], o_dst(c0 - 6), ss[j]).wait()

            pltpu.async_copy(g_src(c0), bufs[j], gs[j])

            @pl.when(g > 0)
            def _(j=j, c1=c1):
                pltpu.make_async_copy(bufs[j + 1], o_dst(c1 - 6), ss[j + 1]).wait()

            pltpu.async_copy(g_src(c1), bufs[j + 1], gs[j + 1])

            @pl.when(g > 0)
            def _(j=j, c2=c2):
                pltpu.make_async_copy(bufs[j + 2], o_dst(c2 - 6), ss[j + 2]).wait()

            expand_group(c2, 0, bufs[j + 2])
            pltpu.make_async_copy(g_src(c0), bufs[j], gs[j]).wait()
            pltpu.async_copy(bufs[j], o_dst(c0), ss[j])
            pltpu.make_async_copy(g_src(c1), bufs[j + 1], gs[j + 1]).wait()
            pltpu.async_copy(bufs[j + 1], o_dst(c1), ss[j + 1])
            pltpu.async_copy(bufs[j + 2], o_dst(c2), ss[j + 2])
        return carry

    lax.fori_loop(0, _NG, body, 0)

    la = _NCH - 2
    lb = _NCH - 1
    pltpu.make_async_copy(bufs[0], o_dst(la - 6), ss[0]).wait()
    pltpu.async_copy(g_src(la), bufs[0], gs[0])
    pltpu.make_async_copy(bufs[1], o_dst(lb - 6), ss[1]).wait()
    pltpu.async_copy(g_src(lb), bufs[1], gs[1])
    for j in range(2, 6):
        pltpu.make_async_copy(bufs[j], o_dst(la - 6 + j), ss[j]).wait()
    pltpu.make_async_copy(g_src(la), bufs[0], gs[0]).wait()
    pltpu.async_copy(bufs[0], o_dst(la), ss[0])
    pltpu.make_async_copy(g_src(lb), bufs[1], gs[1]).wait()
    pltpu.async_copy(bufs[1], o_dst(lb), ss[1])
    pltpu.make_async_copy(bufs[0], o_dst(la), ss[0]).wait()
    pltpu.make_async_copy(bufs[1], o_dst(lb), ss[1]).wait()


def kernel(height_ids, width_ids, h_embed, w_embed):
    hid = height_ids.astype(jnp.int32)
    wid = width_ids.astype(jnp.int32)
    cid, tab = _prep(hid, wid, h_embed, w_embed)
    out = _sc_hybrid(
        tab.reshape(_MAX_H * _MAX_W, _DIM),
        cid.reshape(_N),
        h_embed.reshape(_MAX_H * _DIM),
        w_embed.reshape(_MAX_W * _DIM),
    )
    return out.reshape(_B, _L, _DIM)
